# baseline (speedup 1.0000x reference)
import jax
import jax.numpy as jnp
from jax import lax
from jax.experimental import pallas as pl
from jax.experimental.pallas import tpu as pltpu
from jax.experimental.pallas import tpu_sc as plsc

B, N, D = 16, 4096, 3
S = 9
T = 8
ROWS = B * N // 128
RPB = N // 128


def _tc_err_prefix(p_ref, t_ref, c_ref):
    d = p_ref[...] - t_ref[...]
    sq = d * d
    hp = jax.lax.Precision.HIGHEST
    k = lax.broadcasted_iota(jnp.int32, (3 * 128, 128), 0)
    m = lax.broadcasted_iota(jnp.int32, (3 * 128, 128), 1)
    g = ((k >= 3 * m) & (k < 3 * m + 3)).astype(jnp.float32)
    r2 = lax.dot(sq, g, precision=hp, preferred_element_type=jnp.float32)
    e = jnp.sqrt(r2)
    kk = lax.broadcasted_iota(jnp.int32, (128, 128), 0)
    jj = lax.broadcasted_iota(jnp.int32, (128, 128), 1)
    u = (kk < jj).astype(jnp.float32)
    intra = lax.dot(e, u, precision=hp, preferred_element_type=jnp.float32)
    ii = lax.broadcasted_iota(jnp.int32, (ROWS, ROWS), 0)
    rr = lax.broadcasted_iota(jnp.int32, (ROWS, ROWS), 1)
    lb = (((ii // RPB) == (rr // RPB)) & (rr < ii)).astype(jnp.float32)
    bp = jnp.sum(lax.dot(lb, e, precision=hp,
                         preferred_element_type=jnp.float32),
                 axis=1, keepdims=True)
    c_ref[...] = intra + bp


def _sc_segments(c_hbm, ip_hbm, tp_hbm, part_hbm, cv, iv, tyv, lbuf):
    cid = lax.axis_index("c")
    sid = lax.axis_index("s")

    @pl.when(cid == 0)
    def _work():
        pltpu.sync_copy(c_hbm.at[sid], cv)
        pltpu.sync_copy(ip_hbm.at[sid], iv)
        pltpu.sync_copy(tp_hbm.at[sid], tyv)

        lanes = lax.iota(jnp.int32, 16)
        m8 = lanes < 8
        starts = plsc.load_gather(iv, [jnp.where(m8, lanes, 0)], mask=m8)
        ends = plsc.load_gather(iv, [jnp.where(m8, lanes + 1, 0)], mask=m8)
        cs = plsc.load_gather(cv, [jnp.where(m8, starts, 0)], mask=m8)
        ce = plsc.load_gather(cv, [jnp.where(m8, ends, 0)], mask=m8)
        seglen = (ends - starts).astype(jnp.float32)
        mean = (ce - cs) / jnp.where(m8, seglen, 1.0)

        t8 = lanes * T
        bv = plsc.load_gather(tyv, [jnp.where(m8, t8, 0)], mask=m8)
        bi = jnp.zeros((16,), jnp.int32)
        for j in range(1, T):
            vj = plsc.load_gather(tyv, [jnp.where(m8, t8 + j, 0)], mask=m8)
            upd = vj > bv
            bi = jnp.where(upd, j, bi)
            bv = jnp.where(upd, vj, bv)

        tsum = jnp.zeros((16,), jnp.float32)
        tcnt = jnp.zeros((16,), jnp.float32)
        for t in range(T):
            mt = m8 & (bi == t)
            st = jnp.sum(jnp.where(mt, mean, 0.0))
            ct = jnp.sum(jnp.where(mt, 1.0, 0.0))
            tsum = jnp.where(lanes == t, st, tsum)
            tcnt = jnp.where(lanes == t, ct, tcnt)

        lbuf[pl.ds(0, 16)] = tsum
        lbuf[pl.ds(16, 16)] = tcnt
        pltpu.sync_copy(lbuf, part_hbm.at[sid])


@jax.jit
def kernel(pred, target, indices, indices_type):
    p2 = pred.reshape(ROWS, 3 * 128)
    t2 = target.reshape(ROWS, 3 * 128)
    c = pl.pallas_call(
        _tc_err_prefix,
        out_shape=jax.ShapeDtypeStruct((ROWS, 128), jnp.float32),
    )(p2, t2)
    c2 = c.reshape(B, N)

    starts = indices[:, :-1]
    ends = indices[:, 1:]
    cs = jnp.take_along_axis(c2, starts, axis=1)
    ce = jnp.take_along_axis(c2, ends, axis=1)
    mean = (ce - cs) / (ends - starts).astype(jnp.float32)
    types = jnp.argmax(indices_type[:, :-1, :], axis=-1)
    fm = mean.reshape(-1)
    ft = types.reshape(-1)
    ts = jax.ops.segment_sum(fm, ft, num_segments=T)
    tc = jax.ops.segment_sum(jnp.ones_like(fm), ft, num_segments=T)
    return jnp.where(tc > 0, ts / jnp.maximum(tc, 1.0), 0.0)


# speedup vs baseline: 1.3321x; 1.3321x over previous
import jax
import jax.numpy as jnp
from jax import lax
from jax.experimental import pallas as pl
from jax.experimental.pallas import tpu as pltpu
from jax.experimental.pallas import tpu_sc as plsc

B, N, D = 16, 4096, 3
S = 9
T = 8
ROWS = B * N // 128
RPB = N // 128


def _tc_err_prefix(p_ref, t_ref, c_ref):
    d = p_ref[0] - t_ref[0]
    sq = d * d
    hp = jax.lax.Precision.HIGHEST
    m = sq.T
    ones3 = jnp.ones((1, 3), jnp.float32)
    r2 = lax.dot(ones3, m, precision=hp,
                 preferred_element_type=jnp.float32)
    e = jnp.sqrt(r2).reshape(RPB, 128)
    kk = lax.broadcasted_iota(jnp.int32, (128, 128), 0)
    jj = lax.broadcasted_iota(jnp.int32, (128, 128), 1)
    u = (kk < jj).astype(jnp.float32)
    intra = lax.dot(e, u, precision=hp, preferred_element_type=jnp.float32)
    ii = lax.broadcasted_iota(jnp.int32, (RPB, RPB), 0)
    rr = lax.broadcasted_iota(jnp.int32, (RPB, RPB), 1)
    lb = ((rr < ii)).astype(jnp.float32)
    bp = lax.dot(lb, jnp.sum(e, axis=1, keepdims=True), precision=hp,
                 preferred_element_type=jnp.float32)
    c_ref[...] = (intra + bp).reshape(1, RPB, 128)


def _sc_segments(c_hbm, ip_hbm, tp_hbm, part_hbm, cv, iv, tyv, lbuf):
    cid = lax.axis_index("c")
    sid = lax.axis_index("s")

    @pl.when(cid == 0)
    def _work():
        pltpu.sync_copy(c_hbm.at[sid], cv)
        pltpu.sync_copy(ip_hbm.at[sid], iv)
        pltpu.sync_copy(tp_hbm.at[sid], tyv)

        lanes = lax.iota(jnp.int32, 16)
        m8 = lanes < 8
        starts = plsc.load_gather(iv, [jnp.where(m8, lanes, 0)], mask=m8)
        ends = plsc.load_gather(iv, [jnp.where(m8, lanes + 1, 0)], mask=m8)
        cs = plsc.load_gather(cv, [jnp.where(m8, starts, 0)], mask=m8)
        ce = plsc.load_gather(cv, [jnp.where(m8, ends, 0)], mask=m8)
        seglen = (ends - starts).astype(jnp.float32)
        mean = (ce - cs) / jnp.where(m8, seglen, 1.0)

        t8 = lanes * T
        bv = plsc.load_gather(tyv, [jnp.where(m8, t8, 0)], mask=m8)
        bi = jnp.zeros((16,), jnp.int32)
        for j in range(1, T):
            vj = plsc.load_gather(tyv, [jnp.where(m8, t8 + j, 0)], mask=m8)
            upd = vj > bv
            bi = jnp.where(upd, j, bi)
            bv = jnp.where(upd, vj, bv)

        tsum = jnp.zeros((16,), jnp.float32)
        tcnt = jnp.zeros((16,), jnp.float32)
        for t in range(T):
            mt = m8 & (bi == t)
            st = jnp.sum(jnp.where(mt, mean, 0.0))
            ct = jnp.sum(jnp.where(mt, 1.0, 0.0))
            tsum = jnp.where(lanes == t, st, tsum)
            tcnt = jnp.where(lanes == t, ct, tcnt)

        lbuf[pl.ds(0, 16)] = tsum
        lbuf[pl.ds(16, 16)] = tcnt
        pltpu.sync_copy(lbuf, part_hbm.at[sid])


@jax.jit
def kernel(pred, target, indices, indices_type):
    c2 = pl.pallas_call(
        _tc_err_prefix,
        grid=(B,),
        in_specs=[
            pl.BlockSpec((1, N, D), lambda b: (b, 0, 0)),
            pl.BlockSpec((1, N, D), lambda b: (b, 0, 0)),
        ],
        out_specs=pl.BlockSpec((1, RPB, 128), lambda b: (b, 0, 0)),
        out_shape=jax.ShapeDtypeStruct((B, RPB, 128), jnp.float32),
    )(pred, target).reshape(B, N)

    ip = jnp.zeros((B, 16), jnp.int32).at[:, :S].set(indices)
    tp = jnp.zeros((B, 80), jnp.float32).at[:, :S * T].set(
        indices_type.reshape(B, S * T))

    mesh = plsc.VectorSubcoreMesh(
        core_axis_name="c", subcore_axis_name="s",
        num_cores=2, num_subcores=16)
    sc = pl.kernel(
        _sc_segments,
        out_type=jax.ShapeDtypeStruct((B, 32), jnp.float32),
        mesh=mesh,
        scratch_types=[
            pltpu.VMEM((N,), jnp.float32),
            pltpu.VMEM((16,), jnp.int32),
            pltpu.VMEM((80,), jnp.float32),
            pltpu.VMEM((32,), jnp.float32),
        ],
        compiler_params=pltpu.CompilerParams(needs_layout_passes=False),
    )
    part = sc(c2, ip, tp)
    acc_s = jnp.sum(part[:, :T], axis=0)
    acc_c = jnp.sum(part[:, 16:16 + T], axis=0)
    return jnp.where(acc_c > 0.0, acc_s / jnp.maximum(acc_c, 1.0), 0.0)


# speedup vs baseline: 2.9913x; 2.2456x over previous
import jax
import jax.numpy as jnp
from jax import lax
from jax.experimental import pallas as pl
from jax.experimental.pallas import tpu as pltpu
from jax.experimental.pallas import tpu_sc as plsc

B, N, D = 16, 4096, 3
S = 9
T = 8
RPB = N // 128


def _tc_err_prefix(p_ref, t_ref, c_ref):
    d = p_ref[0] - t_ref[0]
    sq = d * d
    hp = jax.lax.Precision.HIGHEST
    ones3 = jnp.ones((1, D), jnp.float32)
    r2 = lax.dot(ones3, sq, precision=hp,
                 preferred_element_type=jnp.float32)
    e = jnp.sqrt(r2).reshape(RPB, 128)
    kk = lax.broadcasted_iota(jnp.int32, (128, 128), 0)
    jj = lax.broadcasted_iota(jnp.int32, (128, 128), 1)
    u = (kk < jj).astype(jnp.float32)
    intra = lax.dot(e, u, precision=hp, preferred_element_type=jnp.float32)
    ii = lax.broadcasted_iota(jnp.int32, (RPB, RPB), 0)
    rr = lax.broadcasted_iota(jnp.int32, (RPB, RPB), 1)
    lb = (rr < ii).astype(jnp.float32)
    bp = lax.dot(lb, jnp.sum(e, axis=1, keepdims=True), precision=hp,
                 preferred_element_type=jnp.float32)
    c_ref[...] = (intra + bp).reshape(1, RPB, 128)


def _sc_segments(c_hbm, ip_hbm, tp_hbm, part_hbm, cv, iv, tyv, lbuf):
    cid = lax.axis_index("c")
    sid = lax.axis_index("s")

    @pl.when(cid == 0)
    def _work():
        pltpu.sync_copy(c_hbm.at[sid], cv)
        pltpu.sync_copy(ip_hbm.at[sid], iv)
        pltpu.sync_copy(tp_hbm.at[sid], tyv)

        lanes = lax.iota(jnp.int32, 16)
        m8 = lanes < 8
        starts = plsc.load_gather(iv, [jnp.where(m8, lanes, 0)], mask=m8)
        ends = plsc.load_gather(iv, [jnp.where(m8, lanes + 1, 0)], mask=m8)
        zero = jnp.zeros((16,), jnp.int32)
        s_i = jnp.where(m8, starts, 0)
        e_i = jnp.where(m8, ends, 0)
        cs = plsc.load_gather(cv, [s_i >> 7, s_i & 127], mask=m8)
        ce = plsc.load_gather(cv, [e_i >> 7, e_i & 127], mask=m8)
        seglen = (ends - starts).astype(jnp.float32)
        mean = (ce - cs) / jnp.where(m8, seglen, 1.0)

        t8 = lanes * T
        bv = plsc.load_gather(tyv, [jnp.where(m8, t8, 0)], mask=m8)
        bi = zero
        for j in range(1, T):
            vj = plsc.load_gather(tyv, [jnp.where(m8, t8 + j, 0)], mask=m8)
            upd = vj > bv
            bi = jnp.where(upd, j, bi)
            bv = jnp.where(upd, vj, bv)

        tsum = jnp.zeros((16,), jnp.float32)
        tcnt = jnp.zeros((16,), jnp.float32)
        for t in range(T):
            mt = m8 & (bi == t)
            st = jnp.sum(jnp.where(mt, mean, 0.0))
            ct = jnp.sum(jnp.where(mt, 1.0, 0.0))
            tsum = jnp.where(lanes == t, st, tsum)
            tcnt = jnp.where(lanes == t, ct, tcnt)

        lbuf[pl.ds(0, 16)] = tsum
        lbuf[pl.ds(16, 16)] = tcnt
        pltpu.sync_copy(lbuf, part_hbm.at[sid])


@jax.jit
def kernel(pred, target, indices, indices_type):
    pt = pred.transpose(0, 2, 1)
    tt = target.transpose(0, 2, 1)
    c3 = pl.pallas_call(
        _tc_err_prefix,
        grid=(B,),
        in_specs=[
            pl.BlockSpec((1, D, N), lambda b: (b, 0, 0)),
            pl.BlockSpec((1, D, N), lambda b: (b, 0, 0)),
        ],
        out_specs=pl.BlockSpec((1, RPB, 128), lambda b: (b, 0, 0)),
        out_shape=jax.ShapeDtypeStruct((B, RPB, 128), jnp.float32),
    )(pt, tt)

    ip = jnp.zeros((B, 16), jnp.int32).at[:, :S].set(indices)
    tp = jnp.zeros((B, 80), jnp.float32).at[:, :S * T].set(
        indices_type.reshape(B, S * T))

    mesh = plsc.VectorSubcoreMesh(
        core_axis_name="c", subcore_axis_name="s",
        num_cores=2, num_subcores=16)
    sc = pl.kernel(
        _sc_segments,
        out_type=jax.ShapeDtypeStruct((B, 32), jnp.float32),
        mesh=mesh,
        scratch_types=[
            pltpu.VMEM((RPB, 128), jnp.float32),
            pltpu.VMEM((16,), jnp.int32),
            pltpu.VMEM((80,), jnp.float32),
            pltpu.VMEM((32,), jnp.float32),
        ],
        compiler_params=pltpu.CompilerParams(needs_layout_passes=False),
    )
    part = sc(c3, ip, tp)
    acc_s = jnp.sum(part[:, :T], axis=0)
    acc_c = jnp.sum(part[:, 16:16 + T], axis=0)
    return jnp.where(acc_c > 0.0, acc_s / jnp.maximum(acc_c, 1.0), 0.0)


# speedup vs baseline: 3.1980x; 1.0691x over previous
import jax
import jax.numpy as jnp
from jax import lax
from jax.experimental import pallas as pl
from jax.experimental.pallas import tpu as pltpu
from jax.experimental.pallas import tpu_sc as plsc

B, N, D = 16, 4096, 3
S = 9
T = 8
RPB = N // 128


def _tc_err_prefix(p_ref, t_ref, c_ref):
    hp = jax.lax.Precision.HIGHEST
    ones3 = jnp.ones((1, D), jnp.float32)
    kk = lax.broadcasted_iota(jnp.int32, (128, 128), 0)
    jj = lax.broadcasted_iota(jnp.int32, (128, 128), 1)
    u = (kk < jj).astype(jnp.float32)
    ii = lax.broadcasted_iota(jnp.int32, (RPB, RPB), 0)
    rr = lax.broadcasted_iota(jnp.int32, (RPB, RPB), 1)
    lb = (rr < ii).astype(jnp.float32)
    for b in range(B):
        d = p_ref[b] - t_ref[b]
        sq = d * d
        r2 = lax.dot(ones3, sq, precision=hp,
                     preferred_element_type=jnp.float32)
        e = jnp.sqrt(r2).reshape(RPB, 128)
        intra = lax.dot(e, u, precision=hp,
                        preferred_element_type=jnp.float32)
        bp = lax.dot(lb, jnp.sum(e, axis=1, keepdims=True), precision=hp,
                     preferred_element_type=jnp.float32)
        c_ref[b] = intra + bp


def _sc_segments(c_hbm, ip_hbm, tp_hbm, part_hbm, cv, iv, tyv, lbuf):
    cid = lax.axis_index("c")
    sid = lax.axis_index("s")

    @pl.when(cid == 0)
    def _work():
        pltpu.sync_copy(c_hbm.at[sid], cv)
        pltpu.sync_copy(ip_hbm.at[sid], iv)
        pltpu.sync_copy(tp_hbm.at[sid], tyv)

        lanes = lax.iota(jnp.int32, 16)
        m8 = lanes < 8
        starts = plsc.load_gather(iv, [jnp.where(m8, lanes, 0)], mask=m8)
        ends = plsc.load_gather(iv, [jnp.where(m8, lanes + 1, 0)], mask=m8)
        zero = jnp.zeros((16,), jnp.int32)
        s_i = jnp.where(m8, starts, 0)
        e_i = jnp.where(m8, ends, 0)
        cs = plsc.load_gather(cv, [s_i >> 7, s_i & 127], mask=m8)
        ce = plsc.load_gather(cv, [e_i >> 7, e_i & 127], mask=m8)
        seglen = (ends - starts).astype(jnp.float32)
        mean = (ce - cs) / jnp.where(m8, seglen, 1.0)

        t8 = lanes * T
        bv = plsc.load_gather(tyv, [jnp.where(m8, t8, 0)], mask=m8)
        bi = zero
        for j in range(1, T):
            vj = plsc.load_gather(tyv, [jnp.where(m8, t8 + j, 0)], mask=m8)
            upd = vj > bv
            bi = jnp.where(upd, j, bi)
            bv = jnp.where(upd, vj, bv)

        tsum = jnp.zeros((16,), jnp.float32)
        tcnt = jnp.zeros((16,), jnp.float32)
        for t in range(T):
            mt = m8 & (bi == t)
            st = jnp.sum(jnp.where(mt, mean, 0.0))
            ct = jnp.sum(jnp.where(mt, 1.0, 0.0))
            tsum = jnp.where(lanes == t, st, tsum)
            tcnt = jnp.where(lanes == t, ct, tcnt)

        lbuf[pl.ds(0, 16)] = tsum
        lbuf[pl.ds(16, 16)] = tcnt
        pltpu.sync_copy(lbuf, part_hbm.at[sid])


@jax.jit
def kernel(pred, target, indices, indices_type):
    pt = pred.transpose(0, 2, 1)
    tt = target.transpose(0, 2, 1)
    c3 = pl.pallas_call(
        _tc_err_prefix,
        out_shape=jax.ShapeDtypeStruct((B, RPB, 128), jnp.float32),
    )(pt, tt)

    ip = jnp.zeros((B, 16), jnp.int32).at[:, :S].set(indices)
    tp = jnp.zeros((B, 80), jnp.float32).at[:, :S * T].set(
        indices_type.reshape(B, S * T))

    mesh = plsc.VectorSubcoreMesh(
        core_axis_name="c", subcore_axis_name="s",
        num_cores=2, num_subcores=16)
    sc = pl.kernel(
        _sc_segments,
        out_type=jax.ShapeDtypeStruct((B, 32), jnp.float32),
        mesh=mesh,
        scratch_types=[
            pltpu.VMEM((RPB, 128), jnp.float32),
            pltpu.VMEM((16,), jnp.int32),
            pltpu.VMEM((80,), jnp.float32),
            pltpu.VMEM((32,), jnp.float32),
        ],
        compiler_params=pltpu.CompilerParams(needs_layout_passes=False),
    )
    part = sc(c3, ip, tp)
    acc_s = jnp.sum(part[:, :T], axis=0)
    acc_c = jnp.sum(part[:, 16:16 + T], axis=0)
    return jnp.where(acc_c > 0.0, acc_s / jnp.maximum(acc_c, 1.0), 0.0)


# speedup vs baseline: 3.4784x; 1.0877x over previous
import jax
import jax.numpy as jnp
from jax import lax
from jax.experimental import pallas as pl
from jax.experimental.pallas import tpu as pltpu
from jax.experimental.pallas import tpu_sc as plsc

B, N, D = 16, 4096, 3
S = 9
T = 8
RPB = N // 128


def _tc_err_prefix(p_ref, t_ref, c_ref):
    hp = jax.lax.Precision.HIGHEST
    ones3 = jnp.ones((1, D), jnp.float32)
    kk = lax.broadcasted_iota(jnp.int32, (128, 128), 0)
    jj = lax.broadcasted_iota(jnp.int32, (128, 128), 1)
    u = (kk < jj).astype(jnp.float32)
    ii = lax.broadcasted_iota(jnp.int32, (RPB, RPB), 0)
    rr = lax.broadcasted_iota(jnp.int32, (RPB, RPB), 1)
    lb = (rr < ii).astype(jnp.float32)
    for b in range(B):
        d = p_ref[b] - t_ref[b]
        sq = d * d
        r2 = lax.dot(ones3, sq, precision=hp,
                     preferred_element_type=jnp.float32)
        e = jnp.sqrt(r2).reshape(RPB, 128)
        intra = lax.dot(e, u, precision=hp,
                        preferred_element_type=jnp.float32)
        bp = lax.dot(lb, jnp.sum(e, axis=1, keepdims=True), precision=hp,
                     preferred_element_type=jnp.float32)
        c_ref[b] = intra + bp


def _sc_segments(c_hbm, ip_hbm, tp_hbm, part_hbm, cv, iv, tyv, lbuf):
    cid = lax.axis_index("c")
    sid = lax.axis_index("s")

    @pl.when(cid == 0)
    def _work():
        pltpu.sync_copy(c_hbm.at[sid], cv)
        pltpu.sync_copy(ip_hbm.at[sid], iv)
        pltpu.sync_copy(tp_hbm.at[sid], tyv)

        lanes = lax.iota(jnp.int32, 16)
        m8 = lanes < 8
        seg = jnp.where(m8, lanes, 0)
        starts = plsc.load_gather(iv, [seg], mask=m8)
        ends = plsc.load_gather(iv, [jnp.where(m8, lanes + 1, 0)], mask=m8)
        zero = jnp.zeros((16,), jnp.int32)
        s_i = jnp.where(m8, starts, 0)
        e_i = jnp.where(m8, ends, 0)
        cs = plsc.load_gather(cv, [s_i >> 7, s_i & 127], mask=m8)
        ce = plsc.load_gather(cv, [e_i >> 7, e_i & 127], mask=m8)
        seglen = (ends - starts).astype(jnp.float32)
        mean = (ce - cs) / jnp.where(m8, seglen, 1.0)

        bv = plsc.load_gather(tyv, [seg, zero], mask=m8)
        bi = zero
        for j in range(1, T):
            vj = plsc.load_gather(tyv, [seg, zero + j], mask=m8)
            upd = vj > bv
            bi = jnp.where(upd, j, bi)
            bv = jnp.where(upd, vj, bv)

        tsum = jnp.zeros((16,), jnp.float32)
        tcnt = jnp.zeros((16,), jnp.float32)
        for t in range(T):
            mt = m8 & (bi == t)
            st = jnp.sum(jnp.where(mt, mean, 0.0))
            ct = jnp.sum(jnp.where(mt, 1.0, 0.0))
            tsum = jnp.where(lanes == t, st, tsum)
            tcnt = jnp.where(lanes == t, ct, tcnt)

        lbuf[pl.ds(0, 16)] = tsum
        lbuf[pl.ds(16, 16)] = tcnt
        pltpu.sync_copy(lbuf, part_hbm.at[sid])


@jax.jit
def kernel(pred, target, indices, indices_type):
    pt = pred.transpose(0, 2, 1)
    tt = target.transpose(0, 2, 1)
    c3 = pl.pallas_call(
        _tc_err_prefix,
        out_shape=jax.ShapeDtypeStruct((B, RPB, 128), jnp.float32),
    )(pt, tt)

    mesh = plsc.VectorSubcoreMesh(
        core_axis_name="c", subcore_axis_name="s",
        num_cores=2, num_subcores=16)
    sc = pl.kernel(
        _sc_segments,
        out_type=jax.ShapeDtypeStruct((B, 32), jnp.float32),
        mesh=mesh,
        scratch_types=[
            pltpu.VMEM((RPB, 128), jnp.float32),
            pltpu.VMEM((S,), jnp.int32),
            pltpu.VMEM((S, T), jnp.float32),
            pltpu.VMEM((32,), jnp.float32),
        ],
        compiler_params=pltpu.CompilerParams(needs_layout_passes=False),
    )
    part = sc(c3, indices, indices_type)
    acc_s = jnp.sum(part[:, :T], axis=0)
    acc_c = jnp.sum(part[:, 16:16 + T], axis=0)
    return jnp.where(acc_c > 0.0, acc_s / jnp.maximum(acc_c, 1.0), 0.0)


# speedup vs baseline: 4.1037x; 1.1797x over previous
import jax
import jax.numpy as jnp
from jax import lax
from jax.experimental import pallas as pl
from jax.experimental.pallas import tpu as pltpu
from jax.experimental.pallas import tpu_sc as plsc

B, N, D = 16, 4096, 3
S = 9
T = 8
RPB = N // 128


def _tc_err_prefix(p_ref, t_ref, c_ref):
    hp = jax.lax.Precision.DEFAULT
    ones3 = jnp.ones((1, D), jnp.float32)
    kk = lax.broadcasted_iota(jnp.int32, (128, 128), 0)
    jj = lax.broadcasted_iota(jnp.int32, (128, 128), 1)
    u = (kk < jj).astype(jnp.float32)
    ii = lax.broadcasted_iota(jnp.int32, (RPB, RPB), 0)
    rr = lax.broadcasted_iota(jnp.int32, (RPB, RPB), 1)
    lb = (rr < ii).astype(jnp.float32)
    for b in range(B):
        d = p_ref[b] - t_ref[b]
        sq = d * d
        r2 = lax.dot(ones3, sq, precision=hp,
                     preferred_element_type=jnp.float32)
        e = jnp.sqrt(r2).reshape(RPB, 128)
        intra = lax.dot(e, u, precision=hp,
                        preferred_element_type=jnp.float32)
        bp = lax.dot(lb, jnp.sum(e, axis=1, keepdims=True), precision=hp,
                     preferred_element_type=jnp.float32)
        c_ref[b] = intra + bp


def _sc_segments(c_hbm, ip_hbm, tp_hbm, part_hbm, cv, iv, tyv, lbuf):
    cid = lax.axis_index("c")
    sid = lax.axis_index("s")

    @pl.when(cid == 0)
    def _work():
        pltpu.sync_copy(c_hbm.at[sid], cv)
        pltpu.sync_copy(ip_hbm.at[sid], iv)
        pltpu.sync_copy(tp_hbm.at[sid], tyv)

        lanes = lax.iota(jnp.int32, 16)
        m8 = lanes < 8
        seg = jnp.where(m8, lanes, 0)
        starts = plsc.load_gather(iv, [seg], mask=m8)
        ends = plsc.load_gather(iv, [jnp.where(m8, lanes + 1, 0)], mask=m8)
        zero = jnp.zeros((16,), jnp.int32)
        s_i = jnp.where(m8, starts, 0)
        e_i = jnp.where(m8, ends, 0)
        cs = plsc.load_gather(cv, [s_i >> 7, s_i & 127], mask=m8)
        ce = plsc.load_gather(cv, [e_i >> 7, e_i & 127], mask=m8)
        seglen = (ends - starts).astype(jnp.float32)
        mean = (ce - cs) / jnp.where(m8, seglen, 1.0)

        bv = plsc.load_gather(tyv, [seg, zero], mask=m8)
        bi = zero
        for j in range(1, T):
            vj = plsc.load_gather(tyv, [seg, zero + j], mask=m8)
            upd = vj > bv
            bi = jnp.where(upd, j, bi)
            bv = jnp.where(upd, vj, bv)

        tsum = jnp.zeros((16,), jnp.float32)
        tcnt = jnp.zeros((16,), jnp.float32)
        for t in range(T):
            mt = m8 & (bi == t)
            st = jnp.sum(jnp.where(mt, mean, 0.0))
            ct = jnp.sum(jnp.where(mt, 1.0, 0.0))
            tsum = jnp.where(lanes == t, st, tsum)
            tcnt = jnp.where(lanes == t, ct, tcnt)

        lbuf[pl.ds(0, 16)] = tsum
        lbuf[pl.ds(16, 16)] = tcnt
        pltpu.sync_copy(lbuf, part_hbm.at[sid])


@jax.jit
def kernel(pred, target, indices, indices_type):
    pt = pred.transpose(0, 2, 1)
    tt = target.transpose(0, 2, 1)
    c3 = pl.pallas_call(
        _tc_err_prefix,
        out_shape=jax.ShapeDtypeStruct((B, RPB, 128), jnp.float32),
    )(pt, tt)

    mesh = plsc.VectorSubcoreMesh(
        core_axis_name="c", subcore_axis_name="s",
        num_cores=2, num_subcores=16)
    sc = pl.kernel(
        _sc_segments,
        out_type=jax.ShapeDtypeStruct((B, 32), jnp.float32),
        mesh=mesh,
        scratch_types=[
            pltpu.VMEM((RPB, 128), jnp.float32),
            pltpu.VMEM((S,), jnp.int32),
            pltpu.VMEM((S, T), jnp.float32),
            pltpu.VMEM((32,), jnp.float32),
        ],
        compiler_params=pltpu.CompilerParams(needs_layout_passes=False),
    )
    part = sc(c3, indices, indices_type)
    acc_s = jnp.sum(part[:, :T], axis=0)
    acc_c = jnp.sum(part[:, 16:16 + T], axis=0)
    return jnp.where(acc_c > 0.0, acc_s / jnp.maximum(acc_c, 1.0), 0.0)
